# Initial kernel scaffold; baseline (speedup 1.0000x reference)
import jax
import jax.numpy as jnp
from jax.experimental import pallas as pl


def kernel(x, edge_index, W1, b1, W2, b2, W3, b3):
    raise NotImplementedError("write your pallas kernel here")


# speedup vs baseline: 17.3888x; 17.3888x over previous
import functools

import jax
import jax.numpy as jnp
from jax import lax
from jax.experimental import pallas as pl
from jax.experimental.pallas import tpu as pltpu
from jax.experimental.pallas import tpu_sc as plsc

NC = 2
NS = 16
NW = NC * NS
C_PAD = 128


def _best_batch(per: int) -> int:
    for b in range(128, 0, -1):
        if per % b == 0:
            return b
    return 1


@functools.lru_cache(maxsize=None)
def _make_propagate(npad: int, chunks: int, batch: int):
    rows_per_tile = npad // NS
    mesh = plsc.VectorSubcoreMesh(core_axis_name="c", subcore_axis_name="s")

    @functools.partial(
        pl.kernel,
        out_type=jax.ShapeDtypeStruct((NC, npad, C_PAD), jnp.float32),
        mesh=mesh,
        scratch_types=[
            pltpu.VMEM_SHARED((npad, C_PAD), jnp.float32),
            pltpu.VMEM((chunks, batch), jnp.int32),
            pltpu.VMEM((chunks, batch), jnp.int32),
            pltpu.VMEM((batch, C_PAD), jnp.float32),
            pltpu.SemaphoreType.DMA,
        ],
    )
    def prop(h_hbm, src_hbm, dst_hbm, zeros_hbm, out_hbm,
             acc, src_v, dst_v, rows_v, sem):
        cid = lax.axis_index("c")
        sid = lax.axis_index("s")
        wid = sid * NC + cid
        pltpu.sync_copy(zeros_hbm, acc.at[pl.ds(sid * rows_per_tile, rows_per_tile)])
        pltpu.sync_copy(src_hbm.at[wid], src_v)
        pltpu.sync_copy(dst_hbm.at[wid], dst_v)
        plsc.subcore_barrier()

        def chunk(j, carry):
            pltpu.async_copy(h_hbm.at[src_v.at[j]], rows_v, sem).wait()
            pltpu.sync_copy(rows_v, acc.at[dst_v.at[j]], add=True)
            return carry

        lax.fori_loop(0, chunks, chunk, 0)
        plsc.subcore_barrier()
        pltpu.sync_copy(acc.at[pl.ds(sid * rows_per_tile, rows_per_tile)],
                        out_hbm.at[cid, pl.ds(sid * rows_per_tile, rows_per_tile)])

    return prop


@functools.lru_cache(maxsize=None)
def _make_degcount(npad: int, chunks: int, batch: int):
    rows_per_tile = npad // NS
    mesh = plsc.VectorSubcoreMesh(core_axis_name="c", subcore_axis_name="s")

    @functools.partial(
        pl.kernel,
        out_type=jax.ShapeDtypeStruct((NC, npad, C_PAD), jnp.float32),
        mesh=mesh,
        scratch_types=[
            pltpu.VMEM_SHARED((npad, C_PAD), jnp.float32),
            pltpu.VMEM((chunks, batch), jnp.int32),
            pltpu.VMEM((batch, C_PAD), jnp.float32),
        ],
    )
    def degc(dst_hbm, ones_hbm, zeros_hbm, out_hbm, acc, dst_v, ones_v):
        cid = lax.axis_index("c")
        sid = lax.axis_index("s")
        wid = sid * NC + cid
        pltpu.sync_copy(zeros_hbm, acc.at[pl.ds(sid * rows_per_tile, rows_per_tile)])
        pltpu.sync_copy(dst_hbm.at[wid], dst_v)
        pltpu.sync_copy(ones_hbm, ones_v)
        plsc.subcore_barrier()

        def chunk(j, carry):
            pltpu.sync_copy(ones_v, acc.at[dst_v.at[j]], add=True)
            return carry

        lax.fori_loop(0, chunks, chunk, 0)
        plsc.subcore_barrier()
        pltpu.sync_copy(acc.at[pl.ds(sid * rows_per_tile, rows_per_tile)],
                        out_hbm.at[cid, pl.ds(sid * rows_per_tile, rows_per_tile)])

    return degc


def _pad_cols(y, w):
    c = y.shape[-1]
    if c == w:
        return y
    return jnp.concatenate([y, jnp.zeros(y.shape[:-1] + (w - c,), y.dtype)], axis=-1)


def _scale_body(dacc_ref, h_ref, hp_ref, dinv_ref):
    n = h_ref.shape[0]
    d = dacc_ref[...]
    dinv = lax.rsqrt(d[0, :n] + d[1, :n] + 1.0)
    dinv_ref[...] = dinv
    hp_ref[...] = dinv * h_ref[...]


def _mm_body(x_ref, w_ref, o_ref):
    o_ref[...] = jnp.dot(x_ref[...], w_ref[...], preferred_element_type=jnp.float32)


def _mid_body(p_ref, h_ref, dinv_ref, w_ref, b_ref, o_ref):
    n = h_ref.shape[0]
    c_in, c_out = w_ref.shape
    dinv = dinv_ref[...]
    p = p_ref[...]
    pre = dinv[:, :c_in] * (p[0, :n, :c_in] + p[1, :n, :c_in] + h_ref[:, :c_in])
    z = jnp.maximum(pre + b_ref[...], 0.0)
    y = dinv[:, :c_out] * jnp.dot(z, w_ref[...], preferred_element_type=jnp.float32)
    o_ref[...] = _pad_cols(y, o_ref.shape[-1])


def _final_body(p_ref, h_ref, dinv_ref, b_ref, o_ref):
    n, c = o_ref.shape
    dinv = dinv_ref[...]
    p = p_ref[...]
    o_ref[...] = dinv[:, :c] * (p[0, :n, :c] + p[1, :n, :c] + h_ref[:, :c]) + b_ref[...]


def _tc(body, out_shape, *args):
    return pl.pallas_call(body, out_shape=out_shape)(*args)


def kernel(x, edge_index, W1, b1, W2, b2, W3, b3):
    n, _ = x.shape
    e = edge_index.shape[1]
    assert e % NW == 0
    per = e // NW
    batch = _best_batch(per)
    chunks = per // batch
    npad = -(-n // (NS * 8)) * (NS * 8)
    rows_per_tile = npad // NS

    ei = edge_index.astype(jnp.int32)
    src = ei[0].reshape(NW, chunks, batch)
    dst = ei[1].reshape(NW, chunks, batch)

    zeros_pad = jnp.zeros((rows_per_tile, C_PAD), jnp.float32)
    ones_deg = jnp.ones((batch, C_PAD), jnp.float32)
    dacc = _make_degcount(npad, chunks, batch)(dst, ones_deg, zeros_pad)

    prop = _make_propagate(npad, chunks, batch)

    h1 = _tc(_mm_body, jax.ShapeDtypeStruct((n, W1.shape[1]), jnp.float32), x, W1)
    h1p, dinv = pl.pallas_call(
        _scale_body,
        out_shape=(jax.ShapeDtypeStruct((n, C_PAD), jnp.float32),
                   jax.ShapeDtypeStruct((n, C_PAD), jnp.float32)),
    )(dacc, h1)

    P1 = prop(h1p, src, dst, zeros_pad)
    h2p = _tc(_mid_body, jax.ShapeDtypeStruct((n, C_PAD), jnp.float32),
              P1, h1p, dinv, W2, b1)

    P2 = prop(h2p, src, dst, zeros_pad)
    h3p = _tc(_mid_body, jax.ShapeDtypeStruct((n, C_PAD), jnp.float32),
              P2, h2p, dinv, W3, b2)

    P3 = prop(h3p, src, dst, zeros_pad)
    out = _tc(_final_body, jax.ShapeDtypeStruct((n, W3.shape[1]), jnp.float32),
              P3, h3p, dinv, b3)
    return out


# speedup vs baseline: 19.5870x; 1.1264x over previous
import functools

import jax
import jax.numpy as jnp
from jax import lax
from jax.experimental import pallas as pl
from jax.experimental.pallas import tpu as pltpu
from jax.experimental.pallas import tpu_sc as plsc

NC = 2
NS = 16
NW = NC * NS
C_PAD = 128


def _best_batch(per: int) -> int:
    for b in range(128, 0, -8):
        if per % b == 0 and b % 8 == 0:
            return b
    raise ValueError(f"no 8-aligned batch divides {per}")


NBUF = 2


@functools.lru_cache(maxsize=None)
def _make_propagate(npad: int, chunks: int, batch: int):
    rows_per_tile = npad // NS
    per = chunks * batch
    nfull = chunks // NBUF
    rem = chunks % NBUF
    assert nfull >= 2
    mesh = plsc.VectorSubcoreMesh(core_axis_name="c", subcore_axis_name="s")

    @functools.partial(
        pl.kernel,
        out_type=jax.ShapeDtypeStruct((NC, npad, C_PAD), jnp.float32),
        mesh=mesh,
        scratch_types=[
            pltpu.VMEM_SHARED((npad, C_PAD), jnp.float32),
            pltpu.VMEM((per,), jnp.int32),
            pltpu.VMEM((chunks, batch), jnp.int32),
            pltpu.VMEM((NBUF, batch, C_PAD), jnp.float32),
            pltpu.SemaphoreType.DMA((NBUF,)),
            pltpu.SemaphoreType.DMA((NBUF,)),
            pltpu.SemaphoreType.DMA((3,)),
        ],
    )
    def prop(h_hbm, src_hbm, dst_hbm, zeros_hbm, out_hbm,
             acc, src_v, dst_v, rows_v, gsem, ssem, isem):
        cid = lax.axis_index("c")
        sid = lax.axis_index("s")
        wid = sid * NC + cid
        zdst = acc.at[pl.ds(sid * rows_per_tile, rows_per_tile)]
        pltpu.async_copy(zeros_hbm, zdst, isem.at[0])
        pltpu.async_copy(src_hbm.at[wid], src_v, isem.at[1])
        pltpu.async_copy(dst_hbm.at[wid], dst_v, isem.at[2])
        pltpu.make_async_copy(src_hbm.at[wid], src_v, isem.at[1]).wait()

        def sidx(j):
            return src_v.at[pl.ds(j * batch, batch)]

        for b in range(NBUF):
            pltpu.async_copy(h_hbm.at[sidx(b)], rows_v.at[b], gsem.at[b])
        pltpu.make_async_copy(dst_hbm.at[wid], dst_v, isem.at[2]).wait()
        pltpu.make_async_copy(zeros_hbm, zdst, isem.at[0]).wait()
        plsc.subcore_barrier()

        def wait_gather(b):
            pltpu.make_async_copy(h_hbm.at[sidx(0)], rows_v.at[b],
                                  gsem.at[b]).wait()

        def wait_scatter(b, j):
            pltpu.make_async_copy(rows_v.at[b], acc.at[dst_v.at[j]],
                                  ssem.at[b]).wait()

        def blk(jj, carry):
            base = jj * NBUF
            for b in range(NBUF):
                wait_gather(b)
                pltpu.async_copy(rows_v.at[b], acc.at[dst_v.at[base + b]],
                                 ssem.at[b], add=True)
            for b in range(NBUF):
                wait_scatter(b, base + b)
                pltpu.async_copy(h_hbm.at[sidx(base + NBUF + b)],
                                 rows_v.at[b], gsem.at[b])
            return carry

        lax.fori_loop(0, nfull - 1, blk, 0)
        base = (nfull - 1) * NBUF
        for b in range(NBUF):
            wait_gather(b)
            pltpu.async_copy(rows_v.at[b], acc.at[dst_v.at[base + b]],
                             ssem.at[b], add=True)
        for r in range(rem):
            wait_scatter(r, base + r)
            pltpu.async_copy(h_hbm.at[sidx(nfull * NBUF + r)],
                             rows_v.at[r], gsem.at[r])
        for r in range(rem):
            wait_gather(r)
            pltpu.async_copy(rows_v.at[r], acc.at[dst_v.at[nfull * NBUF + r]],
                             ssem.at[r], add=True)
        for b in range(NBUF):
            wait_scatter(b, nfull * NBUF + b if b < rem else base + b)
        plsc.subcore_barrier()
        pltpu.sync_copy(acc.at[pl.ds(sid * rows_per_tile, rows_per_tile)],
                        out_hbm.at[cid, pl.ds(sid * rows_per_tile, rows_per_tile)])

    return prop


@functools.lru_cache(maxsize=None)
def _make_degcount(npad: int, chunks: int, batch: int):
    rows_per_tile = npad // NS
    mesh = plsc.VectorSubcoreMesh(core_axis_name="c", subcore_axis_name="s")

    @functools.partial(
        pl.kernel,
        out_type=jax.ShapeDtypeStruct((NC, npad, C_PAD), jnp.float32),
        mesh=mesh,
        scratch_types=[
            pltpu.VMEM_SHARED((npad, C_PAD), jnp.float32),
            pltpu.VMEM((chunks, batch), jnp.int32),
            pltpu.VMEM((batch, C_PAD), jnp.float32),
            pltpu.SemaphoreType.DMA,
        ],
    )
    def degc(dst_hbm, ones_hbm, zeros_hbm, out_hbm, acc, dst_v, ones_v, sem):
        cid = lax.axis_index("c")
        sid = lax.axis_index("s")
        wid = sid * NC + cid
        pltpu.sync_copy(zeros_hbm, acc.at[pl.ds(sid * rows_per_tile, rows_per_tile)])
        pltpu.sync_copy(dst_hbm.at[wid], dst_v)
        pltpu.sync_copy(ones_hbm, ones_v)
        plsc.subcore_barrier()

        def fire(j, carry):
            pltpu.async_copy(ones_v, acc.at[dst_v.at[j]], sem, add=True)
            return carry

        lax.fori_loop(0, chunks, fire, 0)

        def drain(j, carry):
            pltpu.make_async_copy(ones_v, acc.at[dst_v.at[0]], sem).wait()
            return carry

        lax.fori_loop(0, chunks, drain, 0)
        plsc.subcore_barrier()
        pltpu.sync_copy(acc.at[pl.ds(sid * rows_per_tile, rows_per_tile)],
                        out_hbm.at[cid, pl.ds(sid * rows_per_tile, rows_per_tile)])

    return degc


def _pad_cols(y, w):
    c = y.shape[-1]
    if c == w:
        return y
    return jnp.concatenate([y, jnp.zeros(y.shape[:-1] + (w - c,), y.dtype)], axis=-1)


def _scale_body(dacc_ref, h_ref, hp_ref, dinv_ref):
    n = h_ref.shape[0]
    d = dacc_ref[...]
    dinv = lax.rsqrt(d[0, :n] + d[1, :n] + 1.0)
    dinv_ref[...] = dinv
    hp_ref[...] = dinv * h_ref[...]


def _mm_body(x_ref, w_ref, o_ref):
    o_ref[...] = jnp.dot(x_ref[...], w_ref[...], preferred_element_type=jnp.float32)


def _mid_body(p_ref, h_ref, dinv_ref, w_ref, b_ref, o_ref):
    n = h_ref.shape[0]
    c_in, c_out = w_ref.shape
    dinv = dinv_ref[...]
    p = p_ref[...]
    pre = dinv[:, :c_in] * (p[0, :n, :c_in] + p[1, :n, :c_in] + h_ref[:, :c_in])
    z = jnp.maximum(pre + b_ref[...], 0.0)
    y = dinv[:, :c_out] * jnp.dot(z, w_ref[...], preferred_element_type=jnp.float32)
    o_ref[...] = _pad_cols(y, o_ref.shape[-1])


def _final_body(p_ref, h_ref, dinv_ref, b_ref, o_ref):
    n, c = o_ref.shape
    dinv = dinv_ref[...]
    p = p_ref[...]
    o_ref[...] = dinv[:, :c] * (p[0, :n, :c] + p[1, :n, :c] + h_ref[:, :c]) + b_ref[...]


def _tc(body, out_shape, *args):
    return pl.pallas_call(body, out_shape=out_shape)(*args)


def kernel(x, edge_index, W1, b1, W2, b2, W3, b3):
    n, _ = x.shape
    e = edge_index.shape[1]
    assert e % NW == 0
    per = e // NW
    batch = _best_batch(per)
    chunks = per // batch
    npad = -(-n // (NS * 8)) * (NS * 8)
    rows_per_tile = npad // NS

    ei = edge_index.astype(jnp.int32)
    src = ei[0].reshape(NW, per)
    dst = ei[1].reshape(NW, chunks, batch)

    zeros_pad = jnp.zeros((rows_per_tile, C_PAD), jnp.float32)
    ones_deg = jnp.ones((batch, C_PAD), jnp.float32)
    dacc = _make_degcount(npad, chunks, batch)(dst, ones_deg, zeros_pad)

    prop = _make_propagate(npad, chunks, batch)

    h1 = _tc(_mm_body, jax.ShapeDtypeStruct((n, W1.shape[1]), jnp.float32), x, W1)
    h1p, dinv = pl.pallas_call(
        _scale_body,
        out_shape=(jax.ShapeDtypeStruct((n, C_PAD), jnp.float32),
                   jax.ShapeDtypeStruct((n, C_PAD), jnp.float32)),
    )(dacc, h1)

    P1 = prop(h1p, src, dst, zeros_pad)
    h2p = _tc(_mid_body, jax.ShapeDtypeStruct((n, C_PAD), jnp.float32),
              P1, h1p, dinv, W2, b1)

    P2 = prop(h2p, src, dst, zeros_pad)
    h3p = _tc(_mid_body, jax.ShapeDtypeStruct((n, C_PAD), jnp.float32),
              P2, h2p, dinv, W3, b2)

    P3 = prop(h3p, src, dst, zeros_pad)
    out = _tc(_final_body, jax.ShapeDtypeStruct((n, W3.shape[1]), jnp.float32),
              P3, h3p, dinv, b3)
    return out


# speedup vs baseline: 20.5475x; 1.0490x over previous
import functools

import jax
import jax.numpy as jnp
from jax import lax
from jax.experimental import pallas as pl
from jax.experimental.pallas import tpu as pltpu
from jax.experimental.pallas import tpu_sc as plsc

NC = 2
NS = 16
NW = NC * NS
C_PAD = 128


BATCH = 128
NBUF = 2


@functools.lru_cache(maxsize=None)
def _make_propagate(npad: int, chunks: int):
    rows_per_tile = npad // NS
    nfull = chunks // NBUF
    rem = chunks % NBUF
    nring = 2 * NBUF
    assert nfull >= 2 and chunks >= nring
    mesh = plsc.VectorSubcoreMesh(core_axis_name="c", subcore_axis_name="s")

    @functools.partial(
        pl.kernel,
        out_type=jax.ShapeDtypeStruct((NC, npad, C_PAD), jnp.float32),
        mesh=mesh,
        scratch_types=[
            pltpu.VMEM_SHARED((npad, C_PAD), jnp.float32),
            pltpu.VMEM((nring, BATCH), jnp.int32),
            pltpu.VMEM((chunks, BATCH), jnp.int32),
            pltpu.VMEM((NBUF, BATCH, C_PAD), jnp.float32),
            pltpu.SemaphoreType.DMA((NBUF,)),
            pltpu.SemaphoreType.DMA((NBUF,)),
            pltpu.SemaphoreType.DMA((2,)),
            pltpu.SemaphoreType.DMA((nring,)),
        ],
    )
    def prop(h_hbm, src_hbm, dst_hbm, zeros_hbm, out_hbm,
             acc, src_v, dst_v, rows_v, gsem, ssem, isem, rsem):
        cid = lax.axis_index("c")
        sid = lax.axis_index("s")
        wid = sid * NC + cid
        zdst = acc.at[pl.ds(sid * rows_per_tile, rows_per_tile)]
        pltpu.async_copy(zeros_hbm, zdst, isem.at[0])
        pltpu.async_copy(dst_hbm.at[wid], dst_v, isem.at[1])

        def start_refill(j):
            s = j % nring
            pltpu.async_copy(src_hbm.at[wid, pl.ds(j * BATCH, BATCH)],
                             src_v.at[s], rsem.at[s])

        def wait_refill(j):
            s = j % nring
            pltpu.make_async_copy(src_hbm.at[wid, pl.ds(0, BATCH)],
                                  src_v.at[s], rsem.at[s]).wait()

        def start_gather(b, j):
            pltpu.async_copy(h_hbm.at[src_v.at[j % nring]], rows_v.at[b],
                             gsem.at[b])

        def wait_gather(b):
            pltpu.make_async_copy(h_hbm.at[src_v.at[0]], rows_v.at[b],
                                  gsem.at[b]).wait()

        def start_scatter(b, j):
            pltpu.async_copy(rows_v.at[b], acc.at[dst_v.at[j]],
                             ssem.at[b], add=True)

        def wait_scatter(b, j):
            pltpu.make_async_copy(rows_v.at[b], acc.at[dst_v.at[j]],
                                  ssem.at[b]).wait()

        for s in range(nring):
            start_refill(s)
        wait_refill(0)
        start_gather(0, 0)
        wait_refill(1)
        start_gather(1, 1)
        pltpu.make_async_copy(dst_hbm.at[wid], dst_v, isem.at[1]).wait()
        pltpu.make_async_copy(zeros_hbm, zdst, isem.at[0]).wait()
        plsc.subcore_barrier()

        wait_gather(0)
        start_scatter(0, 0)
        wait_gather(1)
        start_scatter(1, 1)

        def blk(jj, carry):
            j0 = jj * NBUF
            nxt = jnp.minimum(j0 + NBUF, chunks - 1)
            wait_refill(j0)
            wait_scatter(0, j0 - 2)
            start_gather(0, j0)
            wait_refill(j0 + 1)
            wait_scatter(1, j0 - 1)
            start_gather(1, j0 + 1)
            start_refill(nxt)
            start_refill(jnp.minimum(j0 + NBUF + 1, chunks - 1))
            wait_gather(0)
            start_scatter(0, j0)
            wait_gather(1)
            start_scatter(1, j0 + 1)
            return carry

        lax.fori_loop(1, nfull, blk, 0)
        last0 = (nfull - 1) * NBUF
        if rem:
            c = nfull * NBUF
            wait_refill(c)
            wait_scatter(0, last0)
            start_gather(0, c)
            wait_gather(0)
            start_scatter(0, c)
            wait_scatter(1, last0 + 1)
            wait_scatter(0, c)
            wait_refill(chunks - 1)
        else:
            wait_scatter(0, last0)
            wait_scatter(1, last0 + 1)
            wait_refill(chunks - 1)
            wait_refill(chunks - 1)
        plsc.subcore_barrier()
        pltpu.sync_copy(acc.at[pl.ds(sid * rows_per_tile, rows_per_tile)],
                        out_hbm.at[cid, pl.ds(sid * rows_per_tile, rows_per_tile)])

    return prop


@functools.lru_cache(maxsize=None)
def _make_degcount(npad: int, chunks: int):
    rows_per_tile = npad // NS
    mesh = plsc.VectorSubcoreMesh(core_axis_name="c", subcore_axis_name="s")

    @functools.partial(
        pl.kernel,
        out_type=jax.ShapeDtypeStruct((NC, npad, C_PAD), jnp.float32),
        mesh=mesh,
        scratch_types=[
            pltpu.VMEM_SHARED((npad, C_PAD), jnp.float32),
            pltpu.VMEM((chunks, BATCH), jnp.int32),
            pltpu.VMEM((BATCH, C_PAD), jnp.float32),
            pltpu.SemaphoreType.DMA,
        ],
    )
    def degc(dst_hbm, ones_hbm, zeros_hbm, out_hbm, acc, dst_v, ones_v, sem):
        cid = lax.axis_index("c")
        sid = lax.axis_index("s")
        wid = sid * NC + cid
        pltpu.sync_copy(zeros_hbm, acc.at[pl.ds(sid * rows_per_tile, rows_per_tile)])
        pltpu.sync_copy(dst_hbm.at[wid], dst_v)
        pltpu.sync_copy(ones_hbm, ones_v)
        plsc.subcore_barrier()

        def fire(j, carry):
            pltpu.async_copy(ones_v, acc.at[dst_v.at[j]], sem, add=True)
            return carry

        lax.fori_loop(0, chunks, fire, 0)

        def drain(j, carry):
            pltpu.make_async_copy(ones_v, acc.at[dst_v.at[0]], sem).wait()
            return carry

        lax.fori_loop(0, chunks, drain, 0)
        plsc.subcore_barrier()
        pltpu.sync_copy(acc.at[pl.ds(sid * rows_per_tile, rows_per_tile)],
                        out_hbm.at[cid, pl.ds(sid * rows_per_tile, rows_per_tile)])

    return degc


def _pad_cols(y, w):
    c = y.shape[-1]
    if c == w:
        return y
    return jnp.concatenate([y, jnp.zeros(y.shape[:-1] + (w - c,), y.dtype)], axis=-1)


def _mm_scale_body(x_ref, w_ref, dacc_ref, hp_ref, dinv_ref):
    n = x_ref.shape[0]
    d = dacc_ref[...]
    dinv = lax.rsqrt(d[0, :n] + d[1, :n] + 1.0)
    dinv_ref[...] = dinv
    y = jnp.dot(x_ref[...], w_ref[...], preferred_element_type=jnp.float32)
    hp_ref[...] = _pad_cols(dinv[:, :y.shape[1]] * y, hp_ref.shape[-1])


def _mid_body(p_ref, h_ref, dinv_ref, w_ref, b_ref, o_ref):
    n = h_ref.shape[0]
    c_in, c_out = w_ref.shape
    dinv = dinv_ref[...]
    p = p_ref[...]
    pre = dinv[:, :c_in] * (p[0, :n, :c_in] + p[1, :n, :c_in] + h_ref[:, :c_in])
    z = jnp.maximum(pre + b_ref[...], 0.0)
    y = dinv[:, :c_out] * jnp.dot(z, w_ref[...], preferred_element_type=jnp.float32)
    o_ref[...] = _pad_cols(y, o_ref.shape[-1])


def _final_body(p_ref, h_ref, dinv_ref, b_ref, o_ref):
    n, c = o_ref.shape
    dinv = dinv_ref[...]
    p = p_ref[...]
    o_ref[...] = dinv[:, :c] * (p[0, :n, :c] + p[1, :n, :c] + h_ref[:, :c]) + b_ref[...]


def _tc(body, out_shape, *args):
    return pl.pallas_call(body, out_shape=out_shape)(*args)


def kernel(x, edge_index, W1, b1, W2, b2, W3, b3):
    n, _ = x.shape
    e = edge_index.shape[1]
    assert e % NW == 0
    per = e // NW
    chunks = -(-per // BATCH)
    perp = chunks * BATCH
    npad = -(-n // (NS * 8)) * (NS * 8)
    if perp > per and npad == n:
        npad += NS * 8
    rows_per_tile = npad // NS

    ei = edge_index.astype(jnp.int32)
    src = ei[0].reshape(NW, per)
    dst = ei[1].reshape(NW, per)
    if perp > per:
        padn = perp - per
        r = jnp.arange(padn, dtype=jnp.int32)
        src = jnp.concatenate(
            [src, jnp.broadcast_to((r % n)[None], (NW, padn))], axis=1)
        dst = jnp.concatenate(
            [dst, jnp.broadcast_to((n + r % (npad - n))[None], (NW, padn))], axis=1)
    dst = dst.reshape(NW, chunks, BATCH)

    zeros_pad = jnp.zeros((rows_per_tile, C_PAD), jnp.float32)
    ones_deg = jnp.ones((BATCH, C_PAD), jnp.float32)
    dacc = _make_degcount(npad, chunks)(dst, ones_deg, zeros_pad)

    prop = _make_propagate(npad, chunks)

    h1p, dinv = pl.pallas_call(
        _mm_scale_body,
        out_shape=(jax.ShapeDtypeStruct((n, C_PAD), jnp.float32),
                   jax.ShapeDtypeStruct((n, C_PAD), jnp.float32)),
    )(x, W1, dacc)

    P1 = prop(h1p, src, dst, zeros_pad)
    h2p = _tc(_mid_body, jax.ShapeDtypeStruct((n, C_PAD), jnp.float32),
              P1, h1p, dinv, W2, b1)

    P2 = prop(h2p, src, dst, zeros_pad)
    h3p = _tc(_mid_body, jax.ShapeDtypeStruct((n, C_PAD), jnp.float32),
              P2, h2p, dinv, W3, b2)

    P3 = prop(h3p, src, dst, zeros_pad)
    out = _tc(_final_body, jax.ShapeDtypeStruct((n, W3.shape[1]), jnp.float32),
              P3, h3p, dinv, b3)
    return out


# speedup vs baseline: 22.1705x; 1.0790x over previous
import functools

import jax
import jax.numpy as jnp
from jax import lax
from jax.experimental import pallas as pl
from jax.experimental.pallas import tpu as pltpu
from jax.experimental.pallas import tpu_sc as plsc

NC = 2
NS = 16
NW = NC * NS
C_PAD = 128
BATCH = 128
NBUF = 2
DEG_F = 8

_SC_PARAMS = pltpu.CompilerParams(use_tc_tiling_on_sc=False)
_SC_PARAMS_NOLAYOUT = pltpu.CompilerParams(use_tc_tiling_on_sc=False,
                                           needs_layout_passes=False)


@functools.lru_cache(maxsize=None)
def _make_propagate(npad: int, chunks: int, f: int):
    cw = C_PAD // f
    rows_per_tile = npad // NS
    nring = 2 * NBUF
    nfull = chunks // NBUF
    assert chunks % NBUF == 0 and nfull >= 2 and chunks >= nring
    mesh = plsc.VectorSubcoreMesh(core_axis_name="c", subcore_axis_name="s")

    @functools.partial(
        pl.kernel,
        out_type=jax.ShapeDtypeStruct((NC, npad, C_PAD), jnp.float32),
        mesh=mesh,
        scratch_types=[
            pltpu.VMEM_SHARED((npad, C_PAD), jnp.float32),
            pltpu.VMEM((nring, BATCH), jnp.int32),
            pltpu.VMEM((chunks, BATCH), jnp.int32),
            pltpu.VMEM((NBUF, BATCH, cw), jnp.float32),
            pltpu.SemaphoreType.DMA((NBUF,)),
            pltpu.SemaphoreType.DMA((NBUF,)),
            pltpu.SemaphoreType.DMA((2,)),
            pltpu.SemaphoreType.DMA((nring,)),
        ],
        compiler_params=_SC_PARAMS,
    )
    def prop(h_hbm, src_hbm, dst_hbm, zeros_hbm, out_hbm,
             acc, src_v, dst_v, rows_v, gsem, ssem, isem, rsem):
        cid = lax.axis_index("c")
        sid = lax.axis_index("s")
        wid = sid * NC + cid
        hview = h_hbm if f == 1 else h_hbm.reshape(h_hbm.shape[0] * f, cw)
        accv = acc if f == 1 else acc.reshape(npad * f, cw)
        zdst = acc.at[pl.ds(sid * rows_per_tile, rows_per_tile)]
        pltpu.async_copy(zeros_hbm, zdst, isem.at[0])
        pltpu.async_copy(dst_hbm.at[wid], dst_v, isem.at[1])

        def start_refill(j):
            s = j % nring
            pltpu.async_copy(src_hbm.at[wid, pl.ds(j * BATCH, BATCH)],
                             src_v.at[s], rsem.at[s])

        def wait_refill(j):
            s = j % nring
            pltpu.make_async_copy(src_hbm.at[wid, pl.ds(0, BATCH)],
                                  src_v.at[s], rsem.at[s]).wait()

        def start_gather(b, j):
            pltpu.async_copy(hview.at[src_v.at[j % nring]], rows_v.at[b],
                             gsem.at[b])

        def wait_gather(b):
            pltpu.make_async_copy(hview.at[src_v.at[0]], rows_v.at[b],
                                  gsem.at[b]).wait()

        def start_scatter(b, j):
            pltpu.async_copy(rows_v.at[b], accv.at[dst_v.at[j]],
                             ssem.at[b], add=True)

        def wait_scatter(b, j):
            pltpu.make_async_copy(rows_v.at[b], accv.at[dst_v.at[j]],
                                  ssem.at[b]).wait()

        for s in range(nring):
            start_refill(s)
        wait_refill(0)
        start_gather(0, 0)
        wait_refill(1)
        start_gather(1, 1)
        pltpu.make_async_copy(dst_hbm.at[wid], dst_v, isem.at[1]).wait()
        pltpu.make_async_copy(zeros_hbm, zdst, isem.at[0]).wait()
        plsc.subcore_barrier()

        wait_gather(0)
        start_scatter(0, 0)
        wait_gather(1)
        start_scatter(1, 1)

        def blk(jj, carry):
            j0 = jj * NBUF
            wait_refill(j0)
            wait_scatter(0, j0 - 2)
            start_gather(0, j0)
            wait_refill(j0 + 1)
            wait_scatter(1, j0 - 1)
            start_gather(1, j0 + 1)
            start_refill(jnp.minimum(j0 + NBUF, chunks - 1))
            start_refill(jnp.minimum(j0 + NBUF + 1, chunks - 1))
            wait_gather(0)
            start_scatter(0, j0)
            wait_gather(1)
            start_scatter(1, j0 + 1)
            return carry

        lax.fori_loop(1, nfull, blk, 0)
        last0 = (nfull - 1) * NBUF
        wait_scatter(0, last0)
        wait_scatter(1, last0 + 1)
        wait_refill(chunks - 1)
        wait_refill(chunks - 1)
        plsc.subcore_barrier()
        pltpu.sync_copy(acc.at[pl.ds(sid * rows_per_tile, rows_per_tile)],
                        out_hbm.at[cid, pl.ds(sid * rows_per_tile, rows_per_tile)])

    return prop


def _fast_rsqrt(x):
    i = plsc.bitcast(x, jnp.int32)
    y = plsc.bitcast(jnp.int32(0x5F3759DF) - (i >> 1), jnp.float32)
    for _ in range(3):
        y = y * (1.5 - 0.5 * x * y * y)
    return y


@functools.lru_cache(maxsize=None)
def _make_degcount(npad: int, chunks2: int):
    cw = C_PAD // DEG_F
    rows_per_tile = npad // NS
    ngrp = -(-rows_per_tile // 16)
    mesh = plsc.VectorSubcoreMesh(core_axis_name="c", subcore_axis_name="s")

    @functools.partial(
        pl.kernel,
        out_type=jax.ShapeDtypeStruct((npad,), jnp.float32),
        mesh=mesh,
        scratch_types=[
            pltpu.VMEM_SHARED((npad, cw), jnp.float32),
            pltpu.VMEM((chunks2, BATCH), jnp.int32),
            pltpu.VMEM((BATCH, cw), jnp.float32),
            pltpu.VMEM((rows_per_tile, cw), jnp.float32),
            pltpu.VMEM((ngrp * 16,), jnp.float32),
            pltpu.SemaphoreType.DMA,
        ],
        compiler_params=_SC_PARAMS_NOLAYOUT,
    )
    def degc(dst_hbm, out_hbm, acc, dst_v, ones_v, tbuf, obuf, sem):
        sid = lax.axis_index("s")
        pltpu.sync_copy(dst_hbm.at[sid], dst_v)
        ones16 = jnp.ones((cw,), jnp.float32)

        def fill_ones(i, carry):
            ones_v[i, :] = ones16
            return carry

        lax.fori_loop(0, BATCH, fill_ones, 0)
        z16 = jnp.zeros((cw,), jnp.float32)

        def fill_zeros(i, carry):
            tbuf[i, :] = z16
            return carry

        lax.fori_loop(0, rows_per_tile, fill_zeros, 0)
        pltpu.sync_copy(tbuf, acc.at[pl.ds(sid * rows_per_tile, rows_per_tile)])
        plsc.subcore_barrier()

        def fire(j, carry):
            pltpu.async_copy(ones_v, acc.at[dst_v.at[j]], sem, add=True)
            return carry

        lax.fori_loop(0, chunks2, fire, 0)

        def drain(j, carry):
            pltpu.make_async_copy(ones_v, acc.at[dst_v.at[0]], sem).wait()
            return carry

        lax.fori_loop(0, chunks2, drain, 0)
        plsc.subcore_barrier()

        pltpu.sync_copy(acc.at[pl.ds(sid * rows_per_tile, rows_per_tile)], tbuf)
        lanes = lax.iota(jnp.int32, 16)
        zc = jnp.zeros((16,), jnp.int32)

        def conv(g, carry):
            rows = jnp.minimum(g * 16 + lanes, rows_per_tile - 1)
            cnt = plsc.load_gather(tbuf, [rows, zc])
            obuf[pl.ds(g * 16, 16)] = _fast_rsqrt(cnt + 1.0)
            return carry

        lax.fori_loop(0, ngrp, conv, 0)
        pltpu.sync_copy(obuf.at[pl.ds(0, rows_per_tile)],
                        out_hbm.at[pl.ds(sid * rows_per_tile, rows_per_tile)])

    return degc


def _pad_cols(y, w):
    c = y.shape[-1]
    if c == w:
        return y
    return jnp.concatenate([y, jnp.zeros(y.shape[:-1] + (w - c,), y.dtype)], axis=-1)


def _mm_scale_body(x_ref, w_ref, dinv_ref, hp_ref):
    dinv = dinv_ref[...]
    y = jnp.dot(x_ref[...], w_ref[...], preferred_element_type=jnp.float32)
    hp_ref[...] = _pad_cols(dinv[:, :y.shape[1]] * y, hp_ref.shape[-1])


def _mid_body(p_ref, h_ref, dinv_ref, w_ref, b_ref, o_ref):
    n = h_ref.shape[0]
    c_in, c_out = w_ref.shape
    dinv = dinv_ref[...]
    p = p_ref[...]
    pre = dinv[:, :c_in] * (p[0, :n, :c_in] + p[1, :n, :c_in] + h_ref[:, :c_in])
    z = jnp.maximum(pre + b_ref[...], 0.0)
    y = dinv[:, :c_out] * jnp.dot(z, w_ref[...], preferred_element_type=jnp.float32)
    o_ref[...] = _pad_cols(y, o_ref.shape[-1])


def _final_body(p_ref, h_ref, dinv_ref, b_ref, o_ref):
    n, c = o_ref.shape
    dinv = dinv_ref[...]
    p = p_ref[...]
    o_ref[...] = dinv[:, :c] * (p[0, :n, :c] + p[1, :n, :c] + h_ref[:, :c]) + b_ref[...]


def _tc(body, out_shape, *args):
    return pl.pallas_call(body, out_shape=out_shape)(*args)


def kernel(x, edge_index, W1, b1, W2, b2, W3, b3):
    n, _ = x.shape
    e = edge_index.shape[1]
    assert e % NW == 0
    per = e // NW
    chunks = -(-per // BATCH)
    chunks = -(-chunks // 8) * 8
    perp = chunks * BATCH
    npad = -(-n // (NS * 8)) * (NS * 8)
    if perp > per and npad == n:
        npad += NS * 8
    rows_per_tile = npad // NS

    ped = e // NS
    chunks2 = -(-ped // BATCH)
    chunks2 = -(-chunks2 // 8) * 8
    if (perp > per or chunks2 * BATCH > ped) and npad == n:
        npad = n + NS * 8
        rows_per_tile = npad // NS

    ei = edge_index.astype(jnp.int32)
    src = ei[0].reshape(NW, per)
    dst = ei[1].reshape(NW, per)
    dstd = ei[1].reshape(NS, ped)
    if perp > per:
        padn = perp - per
        r = jnp.arange(padn, dtype=jnp.int32)
        src = jnp.concatenate(
            [src, jnp.broadcast_to((r % n)[None], (NW, padn))], axis=1)
        dst = jnp.concatenate(
            [dst, jnp.broadcast_to((n + r % (npad - n))[None], (NW, padn))], axis=1)
    dst = dst.reshape(NW, chunks, BATCH)
    padd = chunks2 * BATCH - ped
    if padd:
        r = jnp.arange(padd, dtype=jnp.int32)
        dstd = jnp.concatenate(
            [dstd, jnp.broadcast_to((n + r % (npad - n))[None], (NS, padd))], axis=1)
    dstd = dstd.reshape(NS, chunks2, BATCH)

    zeros_pad = jnp.zeros((rows_per_tile, C_PAD), jnp.float32)
    dinv1 = _make_degcount(npad, chunks2)(dstd)
    dinv = jnp.broadcast_to(dinv1[:n, None], (n, C_PAD))

    h1p = _tc(_mm_scale_body, jax.ShapeDtypeStruct((n, C_PAD), jnp.float32),
              x, W1, dinv)

    def propagate(h, c_real):
        del c_real
        return _make_propagate(npad, chunks, 1)(h, src, dst, zeros_pad)

    P1 = propagate(h1p, W1.shape[1])
    h2p = _tc(_mid_body, jax.ShapeDtypeStruct((n, C_PAD), jnp.float32),
              P1, h1p, dinv, W2, b1)

    P2 = propagate(h2p, W2.shape[1])
    h3p = _tc(_mid_body, jax.ShapeDtypeStruct((n, C_PAD), jnp.float32),
              P2, h2p, dinv, W3, b2)

    P3 = propagate(h3p, W3.shape[1])
    out = _tc(_final_body, jax.ShapeDtypeStruct((n, W3.shape[1]), jnp.float32),
              P3, h3p, dinv, b3)
    return out
